# Initial kernel scaffold; baseline (speedup 1.0000x reference)
import jax
import jax.numpy as jnp
from jax.experimental import pallas as pl


def kernel(xyz, x, rgb, rgbx, B_xyz, B_rgb, bn_gamma_0, bn_beta_0, bn_gamma_1, bn_beta_1, bn_gamma_2, bn_beta_2, bn_gamma_3, bn_beta_3):
    raise NotImplementedError("write your pallas kernel here")


# speedup vs baseline: 1.2438x; 1.2438x over previous
import functools
import math

import jax
import jax.numpy as jnp
from jax.experimental import pallas as pl

_N0 = 2048
_K = 24
_ED = 144


def _fps_kernel(pts_ref, o_ref, *, npoint):
    pts = pts_ref[0]
    n = pts.shape[1]
    col = jax.lax.broadcasted_iota(jnp.int32, (1, n), 1)
    pcol = jax.lax.broadcasted_iota(jnp.int32, (1, npoint), 1)

    def body(i, state):
        idxs, dists, far = state
        idxs = jnp.where(pcol == i, far, idxs)
        centroid = jnp.sum(jnp.where(col == far, pts, 0.0), axis=1, keepdims=True)
        d = jnp.sum((pts - centroid) ** 2, axis=0, keepdims=True)
        dists = jnp.minimum(dists, d)
        far = jnp.argmax(dists).astype(jnp.int32)
        return idxs, dists, far

    init = (jnp.zeros((1, npoint), jnp.int32), jnp.full((1, n), 1e10, jnp.float32), jnp.int32(0))
    idxs, _, _ = jax.lax.fori_loop(0, npoint, body, init)
    o_ref[0] = idxs


def _fps(xyz, npoint):
    b, n, _ = xyz.shape
    xt3 = jnp.transpose(xyz, (0, 2, 1))
    out = pl.pallas_call(
        functools.partial(_fps_kernel, npoint=npoint),
        grid=(b,),
        in_specs=[pl.BlockSpec((1, 3, n), lambda i: (i, 0, 0))],
        out_specs=pl.BlockSpec((1, 1, npoint), lambda i: (i, 0, 0)),
        out_shape=jax.ShapeDtypeStruct((b, 1, npoint), jnp.int32),
    )(xt3)
    return out[:, 0, :]


def _stage_kernel(kxyz_ref, krgb_ref, kx_ref, lxyz_ref, lrgb_ref, lx_ref,
                  bx_ref, br_ref, st_ref, o_ref, *, fd):
    std_x = st_ref[0, 0]
    std_xyz = st_ref[0, 1]
    std_rgb = st_ref[0, 2]
    lx = lx_ref[0]
    gt, k, _ = kx_ref.shape[1], kx_ref.shape[2], kx_ref.shape[3]
    nx = (kx_ref[0] - lx[:, None, :]) / (std_x + 1e-5)
    bx = bx_ref[0, :fd]
    br = br_ref[0, :fd]
    twopi = 2.0 * math.pi
    for d3 in range(3):
        tx = (kxyz_ref[0, :, :, d3] - lxyz_ref[0, :, d3][:, None]) / (std_xyz + 1e-5)
        tr = (krgb_ref[0, :, :, d3] - lrgb_ref[0, :, d3][:, None]) / (std_rgb + 1e-5)
        divx = (twopi * tx)[:, :, None] * bx[None, None, :]
        divr = (twopi * tr)[:, :, None] * br[None, None, :]
        ex = jnp.concatenate([jnp.sin(divx), jnp.cos(divx)], axis=-1)
        er = jnp.concatenate([jnp.sin(divr), jnp.cos(divr)], axis=-1)
        if d3 == 0:
            f = nx[:, :, 0:2 * fd]
        elif d3 == 1:
            f = jnp.concatenate(
                [nx[:, :, 2 * fd:3 * fd],
                 jnp.broadcast_to(lx[:, None, 0:fd], (gt, k, fd))], axis=-1)
        else:
            f = jnp.broadcast_to(lx[:, None, fd:3 * fd], (gt, k, 2 * fd))
        a = f + ex
        w = 0.8 * (a * ex) + 0.2 * (a * er)
        o_ref[0, :, 2 * fd * d3:2 * fd * (d3 + 1)] = (
            jnp.max(w, axis=1) + jnp.mean(w, axis=1))


def _stage(knn_xyz, knn_rgb, knn_x, lc_xyz, lc_rgb, lc_x, b_xyz, b_rgb,
           stats, fd, gt):
    b, g = knn_x.shape[0], knn_x.shape[1]
    d = knn_x.shape[3]
    out_dim = 6 * fd
    return pl.pallas_call(
        functools.partial(_stage_kernel, fd=fd),
        grid=(b, g // gt),
        in_specs=[
            pl.BlockSpec((1, gt, _K, 3), lambda i, j: (i, j, 0, 0)),
            pl.BlockSpec((1, gt, _K, 3), lambda i, j: (i, j, 0, 0)),
            pl.BlockSpec((1, gt, _K, d), lambda i, j: (i, j, 0, 0)),
            pl.BlockSpec((1, gt, 3), lambda i, j: (i, j, 0)),
            pl.BlockSpec((1, gt, 3), lambda i, j: (i, j, 0)),
            pl.BlockSpec((1, gt, d), lambda i, j: (i, j, 0)),
            pl.BlockSpec((1, 384), lambda i, j: (0, 0)),
            pl.BlockSpec((1, 384), lambda i, j: (0, 0)),
            pl.BlockSpec((1, 3), lambda i, j: (0, 0)),
        ],
        out_specs=pl.BlockSpec((1, gt, out_dim), lambda i, j: (i, j, 0)),
        out_shape=jax.ShapeDtypeStruct((b, g, out_dim), jnp.float32),
    )(knn_xyz, knn_rgb, knn_x, lc_xyz, lc_rgb, lc_x, b_xyz, b_rgb, stats)


def _bn_kernel(t_ref, g_ref, b_ref, o_ref):
    t = t_ref[...]
    mean = jnp.mean(t, axis=(0, 1), keepdims=True)
    var = jnp.mean((t - mean) ** 2, axis=(0, 1), keepdims=True)
    th = (t - mean) / jnp.sqrt(var + 1e-5)
    th = th * g_ref[0][None, None, :] + b_ref[0][None, None, :]
    o_ref[...] = 0.5 * th * (1.0 + jax.lax.erf(th / math.sqrt(2.0)))


def _bn(t, gamma, beta):
    b, g, c = t.shape
    return pl.pallas_call(
        _bn_kernel,
        out_shape=jax.ShapeDtypeStruct((b, g, c), jnp.float32),
    )(t, gamma.reshape(1, c), beta.reshape(1, c))


def _take(p, i):
    return jax.vmap(lambda pp, ii: pp[ii])(p, i)


def kernel(xyz, x, rgb, rgbx, B_xyz, B_rgb, bn_gamma_0, bn_beta_0, bn_gamma_1,
           bn_beta_1, bn_gamma_2, bn_beta_2, bn_gamma_3, bn_beta_3):
    gammas = [bn_gamma_0, bn_gamma_1, bn_gamma_2, bn_gamma_3]
    betas = [bn_beta_0, bn_beta_1, bn_beta_2, bn_beta_3]
    bb, n0, _ = xyz.shape

    fd0 = _ED // 6
    fr = jnp.arange(fd0, dtype=jnp.float32) / fd0
    de = jnp.power(1000.0, fr)
    div = 100.0 * x[..., None] / de
    e = jnp.stack([jnp.sin(div), jnp.cos(div)], axis=4).reshape(bb, 3, n0, 2 * fd0)
    x_emb = jnp.transpose(e, (0, 1, 3, 2)).reshape(bb, _ED, n0)

    xyz_list = [xyz]
    x_list = [x_emb]
    xt = jnp.transpose(x_emb, (0, 2, 1))
    cur_xyz = xyz
    cur_rgb = rgb
    out_dim = _ED
    group = n0
    gts = [128, 64, 32, 16]
    for i in range(4):
        out_dim *= 2
        group //= 2
        fd = out_dim // 6
        fps_idx = _fps(cur_xyz, group)
        lc_xyz = _take(cur_xyz, fps_idx)
        lc_x = _take(xt, fps_idx)
        lc_rgb = _take(cur_rgb, fps_idx)
        d = jnp.sum((lc_xyz[:, :, None, :] - cur_xyz[:, None, :, :]) ** 2, axis=-1)
        _, knn_idx = jax.lax.top_k(-d, _K)
        knn_xyz = _take(cur_xyz, knn_idx)
        knn_x = _take(xt, knn_idx)
        knn_rgb = _take(cur_rgb, knn_idx)
        std_x = jnp.std(knn_x - lc_x[:, :, None, :], ddof=1)
        std_xyz = jnp.std(knn_xyz - lc_xyz[:, :, None, :], ddof=1)
        std_rgb = jnp.std(knn_rgb - lc_rgb[:, :, None, :], ddof=1)
        stats = jnp.stack([std_x, std_xyz, std_rgb])[None, :]
        pooled = _stage(knn_xyz, knn_rgb, knn_x, lc_xyz, lc_rgb, lc_x,
                        B_xyz, B_rgb, stats, fd, gts[i])
        xt = _bn(pooled, gammas[i], betas[i])
        x_out = jnp.transpose(xt, (0, 2, 1))
        xyz_list.append(lc_xyz)
        x_list.append(x_out)
        cur_xyz = lc_xyz
        cur_rgb = lc_rgb
    return tuple(xyz_list), tuple(x_list)
